# Initial kernel scaffold; baseline (speedup 1.0000x reference)
import jax
import jax.numpy as jnp
from jax.experimental import pallas as pl


def kernel(qkv, sim, x_size, logit_scale, W_proj, b_proj):
    raise NotImplementedError("write your pallas kernel here")


# speedup vs baseline: 2232.5022x; 2232.5022x over previous
import functools
import math

import jax
import jax.numpy as jnp
from jax import lax
from jax.experimental import pallas as pl
from jax.experimental.pallas import tpu as pltpu
from jax.experimental.pallas import tpu_sc as plsc

NUM_HEADS = 12
B = 8
N = 1024
DIM = 768
HD = DIM // NUM_HEADS
GS = 128
NG = N // GS
NTOK = B * N
NSIM = 64

NC = 2
NS = 16
NW = NC * NS
RPW = NTOK // NW

CH_C = 16
NCH_C = RPW // CH_C
CH_D = 32
NCH_D = RPW // CH_D

_MAXLOG = math.log(1.0 / 0.01)


def _rank_body(sim_ref, simt_ref, rev_ref):
    b = pl.program_id(0)
    s = sim_ref[0]
    st = simt_ref[0]

    m2 = jnp.max(s, axis=1, keepdims=True)
    col2 = lax.broadcasted_iota(jnp.int32, (N, NSIM), 1)
    tk2 = jnp.min(jnp.where(s == m2, col2, NSIM), axis=1, keepdims=True)

    mt = jnp.max(st, axis=0, keepdims=True)
    colt = lax.broadcasted_iota(jnp.int32, (NSIM, N), 0)
    tkt = jnp.min(jnp.where(st == mt, colt, NSIM), axis=0, keepdims=True)

    key2 = tk2 * N + lax.broadcasted_iota(jnp.int32, (N, 1), 0)
    keyt = tkt * N + lax.broadcasted_iota(jnp.int32, (1, N), 1)

    cmp = (keyt > key2).astype(jnp.float32)
    rank = jnp.sum(cmp, axis=0, keepdims=True)
    rev_ref[...] = (rank.astype(jnp.int32) + b * N).reshape(1, 1, N)


def _rank_call(sim, simt):
    return pl.pallas_call(
        _rank_body,
        grid=(B,),
        in_specs=[
            pl.BlockSpec((1, N, NSIM), lambda b: (b, 0, 0)),
            pl.BlockSpec((1, NSIM, N), lambda b: (b, 0, 0)),
        ],
        out_specs=pl.BlockSpec((1, 1, N), lambda b: (b, 0, 0)),
        out_shape=jax.ShapeDtypeStruct((B, 1, N), jnp.int32),
    )(sim, simt)


def _sc_scatter_call(qkv_flat, idx3):
    mesh = plsc.VectorSubcoreMesh(core_axis_name="c", subcore_axis_name="s")

    @functools.partial(
        pl.kernel,
        mesh=mesh,
        out_type=jax.ShapeDtypeStruct((NTOK, 3 * DIM), jnp.float32),
        scratch_types=[
            pltpu.VMEM((NCH_C, CH_C), jnp.int32),
            pltpu.VMEM((CH_C, 3 * DIM), jnp.float32),
            pltpu.SemaphoreType.DMA,
        ],
    )
    def k(qkv_hbm, idx_hbm, out_hbm, idx_v, buf, sem):
        cid = lax.axis_index("c")
        sid = lax.axis_index("s")
        w = sid * NC + cid
        base = w * RPW
        pltpu.sync_copy(idx_hbm.at[w], idx_v)
        for j in range(NCH_C):
            pltpu.sync_copy(qkv_hbm.at[pl.ds(base + j * CH_C, CH_C)], buf)
            pltpu.async_copy(buf, out_hbm.at[idx_v.at[j]], sem).wait()

    return k(qkv_flat, idx3)


def _attn_body(ls_ref, x_ref, w_ref, b_ref, o_ref):
    sc = jnp.exp(jnp.minimum(ls_ref[0, 0], _MAXLOG))
    x = x_ref[0]
    outs = []
    for h in range(NUM_HEADS):
        q = x[:, h * HD:(h + 1) * HD]
        kk = x[:, DIM + h * HD:DIM + (h + 1) * HD]
        v = x[:, 2 * DIM + h * HD:2 * DIM + (h + 1) * HD]
        a = lax.dot_general(q, kk, (((1,), (1,)), ((), ())),
                            preferred_element_type=jnp.float32)
        a = a * sc
        a = a - jnp.max(a, axis=-1, keepdims=True)
        e = jnp.exp(a)
        p = e / jnp.sum(e, axis=-1, keepdims=True)
        outs.append(lax.dot_general(p, v, (((1,), (0,)), ((), ())),
                                    preferred_element_type=jnp.float32))
    y = jnp.concatenate(outs, axis=1)
    o_ref[0] = lax.dot_general(y, w_ref[...], (((1,), (1,)), ((), ())),
                               preferred_element_type=jnp.float32) + b_ref[...]


def _attn_call(logit_scale, shuf3, W_proj, b2):
    return pl.pallas_call(
        _attn_body,
        grid=(B * NG,),
        in_specs=[
            pl.BlockSpec((1, 1), lambda i: (0, 0)),
            pl.BlockSpec((1, GS, 3 * DIM), lambda i: (i, 0, 0)),
            pl.BlockSpec((DIM, DIM), lambda i: (0, 0)),
            pl.BlockSpec((1, DIM), lambda i: (0, 0)),
        ],
        out_specs=pl.BlockSpec((1, GS, DIM), lambda i: (i, 0, 0)),
        out_shape=jax.ShapeDtypeStruct((B * NG, GS, DIM), jnp.float32),
    )(logit_scale, shuf3, W_proj, b2)


def _sc_gather_call(src_flat, idx3):
    mesh = plsc.VectorSubcoreMesh(core_axis_name="c", subcore_axis_name="s")

    @functools.partial(
        pl.kernel,
        mesh=mesh,
        out_type=jax.ShapeDtypeStruct((NTOK, DIM), jnp.float32),
        scratch_types=[
            pltpu.VMEM((NCH_D, CH_D), jnp.int32),
            pltpu.VMEM((CH_D, DIM), jnp.float32),
            pltpu.SemaphoreType.DMA,
        ],
    )
    def k(src_hbm, idx_hbm, out_hbm, idx_v, buf, sem):
        cid = lax.axis_index("c")
        sid = lax.axis_index("s")
        w = sid * NC + cid
        base = w * RPW
        pltpu.sync_copy(idx_hbm.at[w], idx_v)
        for j in range(NCH_D):
            pltpu.async_copy(src_hbm.at[idx_v.at[j]], buf, sem).wait()
            pltpu.sync_copy(buf, out_hbm.at[pl.ds(base + j * CH_D, CH_D)])

    return k(src_flat, idx3)


def kernel(qkv, sim, x_size, logit_scale, W_proj, b_proj):
    del x_size
    qkv_flat = qkv.reshape(NTOK, 3 * DIM)
    simt = jnp.swapaxes(sim, 1, 2)
    rev = _rank_call(sim, simt).reshape(NTOK)

    shuf = _sc_scatter_call(qkv_flat, rev.reshape(NW, NCH_C, CH_C))
    y = _attn_call(logit_scale, shuf.reshape(B * NG, GS, 3 * DIM),
                   W_proj, b_proj.reshape(1, DIM))
    out = _sc_gather_call(y.reshape(NTOK, DIM), rev.reshape(NW, NCH_D, CH_D))
    return out.reshape(B, N, DIM)
